# baseline (speedup 1.0000x reference)
import functools

import jax
import jax.numpy as jnp
from jax import lax
from jax.experimental import pallas as pl
from jax.experimental.pallas import tpu as pltpu
from jax.experimental.pallas import tpu_sc as plsc

NC = 2
NS = 16
NW = NC * NS
LANES = 128
B = 32
EPW = 5120


def _matmul_chunked(A, W, b, bm, out_dtype=jnp.float32):
    M, K = A.shape
    Hout = W.shape[1]
    C = Hout // LANES
    assert Hout % LANES == 0 and M % bm == 0

    def body(a_ref, w_ref, b_ref, o_ref):
        o_ref[0] = (
            jnp.dot(a_ref[...], w_ref[...], preferred_element_type=jnp.float32)
            + b_ref[...]
        ).astype(out_dtype)

    return pl.pallas_call(
        body,
        grid=(M // bm, C),
        in_specs=[
            pl.BlockSpec((bm, K), lambda i, j: (i, 0)),
            pl.BlockSpec((K, LANES), lambda i, j: (0, j)),
            pl.BlockSpec((1, LANES), lambda i, j: (0, j)),
        ],
        out_specs=pl.BlockSpec((1, bm, LANES), lambda i, j: (j, i, 0)),
        out_shape=jax.ShapeDtypeStruct((C, M, LANES), out_dtype),
    )(A, W, b.reshape(1, Hout))


def _pack_i32(x):
    return jax.lax.bitcast_convert_type(
        x.reshape(*x.shape[:-1], x.shape[-1] // 2, 2), jnp.int32)


def _unpack_perm(hout):
    import numpy as np
    p = np.arange(hout)
    g, r = p // 32, p % 32
    return np.where(r % 2 == 0, 32 * g + r // 2, 32 * g + 16 + r // 2)


def _sc_edge_pass(src, dst, proj, pe, zeros_h, n_pad, with_deg):
    E = src.shape[0]
    C = pe.shape[0]
    epw = E // NW
    nb = epw // B
    nb2 = nb // 2
    rps = n_pad // NS
    assert E % NW == 0 and epw % B == 0 and nb % 2 == 0 and B % 16 == 0

    out_type = [jax.ShapeDtypeStruct((NC, C, n_pad, LANES), jnp.float32)]
    if with_deg:
        out_type.append(jax.ShapeDtypeStruct((NC, n_pad, LANES), jnp.float32))
    scratch = [
        pltpu.VMEM((epw,), jnp.int32),
        pltpu.VMEM((epw,), jnp.int32),
        pltpu.VMEM((B,), jnp.int32),
        pltpu.VMEM((B,), jnp.int32),
        pltpu.VMEM((B, LANES), jnp.float32),
        pltpu.VMEM((B, LANES), jnp.float32),
        pltpu.VMEM((B, LANES), jnp.float32),
        pltpu.VMEM((B, LANES), jnp.float32),
        pltpu.VMEM((B, LANES), jnp.float32),
        pltpu.VMEM((B, LANES), jnp.float32),
        pltpu.VMEM((B, LANES), jnp.float32),
        pltpu.VMEM((B, LANES), jnp.float32),
        pltpu.VMEM_SHARED((n_pad, LANES), jnp.float32),
    ] + [pltpu.SemaphoreType.DMA] * 8

    @functools.partial(
        pl.kernel,
        mesh=plsc.VectorSubcoreMesh(core_axis_name="c", subcore_axis_name="s"),
        out_type=out_type,
        scratch_types=scratch,
    )
    def k(src_h, dst_h, *rest):
        projs = rest[:2 * C]
        pes_h = rest[2 * C:3 * C]
        zz_h = rest[3 * C]
        rest = rest[3 * C + 1:]
        if with_deg:
            agg_h, deg_h = rest[0], rest[1]
            rest = rest[2:]
        else:
            agg_h = rest[0]
            rest = rest[1:]
        (sidx_all, didx_all, db0, db1, gs0, gs1, gd0, gd1, pe0, pe1,
         ms0, ms1, acc, gsm0, gsm1, dsm0, dsm1, psm0, psm1,
         ssm0, ssm1) = rest
        didx_buf = (db0, db1)
        gs = (gs0, gs1)
        gd = (gd0, gd1)
        pes = (pe0, pe1)
        msg = (ms0, ms1)
        gsm = (gsm0, gsm1)
        dsm = (dsm0, dsm1)
        psm = (psm0, psm1)
        ssm = (ssm0, ssm1)

        c = lax.axis_index("c")
        s = lax.axis_index("s")
        w = s * NC + c
        ebase = w * epw
        row0 = s * rps

        pltpu.sync_copy(src_h.at[pl.ds(ebase, epw)], sidx_all)
        pltpu.sync_copy(dst_h.at[pl.ds(ebase, epw)], didx_all)

        npass = C + 1 if with_deg else C
        for ch in range(npass):
            deg_pass = ch == C
            pltpu.sync_copy(zz_h.at[pl.ds(row0, rps)],
                            acc.at[pl.ds(row0, rps)])
            plsc.subcore_barrier()

            if deg_pass:
                def fill1(i, _):
                    for r in range(LANES // 16):
                        ms0[i, pl.ds(r * 16, 16)] = jnp.ones(
                            (16,), jnp.float32)
                    return 0
                lax.fori_loop(0, B, fill1, 0)

                def dbatch(i, _):
                    off = i * B
                    for r in range(B // 16):
                        db0[pl.ds(r * 16, 16)] = (
                            didx_all[pl.ds(off + r * 16, 16)])
                    pltpu.sync_copy(ms0, acc.at[db0], add=True)
                    return 0
                lax.fori_loop(0, nb, dbatch, 0)
            else:
                ps_h = projs[ch]
                pd_h = projs[C + ch]
                pe_ch = pes_h[ch]

                def issue(b, t):
                    off = b * B
                    pltpu.async_copy(
                        ps_h.at[sidx_all.at[pl.ds(off, B)]], gs[t], gsm[t])
                    pltpu.async_copy(
                        pd_h.at[didx_all.at[pl.ds(off, B)]], gd[t], dsm[t])
                    pltpu.async_copy(
                        pe_ch.at[pl.ds(ebase + off, B)], pes[t], psm[t])

                def wait_in(t):
                    pltpu.make_async_copy(
                        ps_h.at[pl.ds(0, B)], gs[t], gsm[t]).wait()
                    pltpu.make_async_copy(
                        pd_h.at[pl.ds(0, B)], gd[t], dsm[t]).wait()
                    pltpu.make_async_copy(
                        pe_ch.at[pl.ds(0, B)], pes[t], psm[t]).wait()

                def wait_sc(t):
                    pltpu.make_async_copy(
                        msg[t], acc.at[pl.ds(0, B)], ssm[t]).wait()

                issue(0, 0)
                issue(1, 1)

                def body2(kk, _):
                    for t in range(2):
                        b = 2 * kk + t
                        off = b * B
                        wait_in(t)
                        for r in range(B // 16):
                            didx_buf[t][pl.ds(r * 16, 16)] = (
                                didx_all[pl.ds(off + r * 16, 16)])

                        @pl.when(kk > 0)
                        def _():
                            wait_sc(t)

                        def comp(j, _):
                            for r in range(LANES // 16):
                                sl = pl.ds(r * 16, 16)
                                msg[t][j, sl] = jnp.maximum(
                                    gs[t][j, sl] + gd[t][j, sl]
                                    + pes[t][j, sl], 0.0)
                            return 0
                        lax.fori_loop(0, B, comp, 0)
                        pltpu.async_copy(
                            msg[t], acc.at[didx_buf[t]], ssm[t], add=True)

                        @pl.when(kk + 1 < nb2)
                        def _():
                            issue(b + 2, t)
                    return 0
                lax.fori_loop(0, nb2, body2, 0)
                wait_sc(0)
                wait_sc(1)

            plsc.subcore_barrier()

            for core in range(NC):
                @pl.when(c == core)
                def _():
                    if deg_pass:
                        pltpu.sync_copy(
                            acc.at[pl.ds(row0, rps)],
                            deg_h.at[core, pl.ds(row0, rps)])
                    else:
                        pltpu.sync_copy(
                            acc.at[pl.ds(row0, rps)],
                            agg_h.at[core, ch, pl.ds(row0, rps)])
            if ch + 1 < npass:
                plsc.subcore_barrier()

    proj_args = [proj[k_] for k_ in range(2 * C)]
    pe_args = [pe[k_] for k_ in range(C)]
    outs = k(src, dst, *proj_args, *pe_args, zeros_h)
    return outs if with_deg else (outs[0] if isinstance(outs, (list, tuple)) else outs)


def _update(h, agg, deg, W_u, b_u, bm, W_out=None, b_out=None):
    M, Din = h.shape
    C = agg.shape[1]
    H = C * LANES
    final = W_out is not None

    def body(*refs):
        if final:
            h_ref, agg_ref, deg_ref, wu_ref, bu_ref, wo_ref, bo_ref, o_ref = refs
        else:
            h_ref, agg_ref, deg_ref, wu_ref, bu_ref, o_ref = refs
        acc = jnp.dot(h_ref[...], wu_ref[pl.ds(0, Din), :],
                      preferred_element_type=jnp.float32)
        deg = deg_ref[0, :, 0:1] + deg_ref[1, :, 0:1]
        inv = 1.0 / jnp.maximum(deg, 1.0)
        for ch in range(C):
            a = (agg_ref[0, ch] + agg_ref[1, ch]) * inv
            acc += jnp.dot(a, wu_ref[pl.ds(Din + ch * LANES, LANES), :],
                           preferred_element_type=jnp.float32)
        hn = jnp.maximum(acc + bu_ref[...], 0.0)
        if final:
            o_ref[...] = (
                jnp.dot(hn, wo_ref[...], preferred_element_type=jnp.float32)
                + bo_ref[...]
            )
        else:
            o_ref[...] = hn

    Dtot = Din + H
    Hout = W_out.shape[1] if final else W_u.shape[1]
    in_specs = [
        pl.BlockSpec((bm, Din), lambda i: (i, 0)),
        pl.BlockSpec((2, C, bm, LANES), lambda i: (0, 0, i, 0)),
        pl.BlockSpec((2, bm, LANES), lambda i: (0, i, 0)),
        pl.BlockSpec((Dtot, W_u.shape[1]), lambda i: (0, 0)),
        pl.BlockSpec((1, W_u.shape[1]), lambda i: (0, 0)),
    ]
    args = [h, agg, deg, W_u, b_u.reshape(1, -1)]
    if final:
        in_specs += [
            pl.BlockSpec((H, Hout), lambda i: (0, 0)),
            pl.BlockSpec((1, Hout), lambda i: (0, 0)),
        ]
        args += [W_out, b_out.reshape(1, -1)]

    return pl.pallas_call(
        body,
        grid=(M // bm,),
        in_specs=in_specs,
        out_specs=pl.BlockSpec((bm, Hout), lambda i: (i, 0)),
        out_shape=jax.ShapeDtypeStruct((M, Hout), jnp.float32),
    )(*args)


@jax.jit
def kernel(x, edge_index, edge_attr,
           W_msg1, b_msg1, W_upd1, b_upd1,
           W_msg2, b_msg2, W_upd2, b_upd2,
           W_out, b_out):
    N, D = x.shape
    E = edge_index.shape[1]
    H = W_msg1.shape[1]
    C = H // LANES
    n_pad = ((N + NS * 128 - 1) // (NS * 128)) * (NS * 128)

    src = edge_index[0]
    dst = edge_index[1]
    hp = jnp.pad(x, ((0, n_pad - N), (0, 0)))
    zeros_h = jnp.zeros((n_pad, LANES), jnp.float32)

    epw0 = E // NW
    DE = edge_attr.shape[1]
    src_p = jnp.full((NW, EPW), N, jnp.int32)
    src_p = src_p.at[:, :epw0].set(src.reshape(NW, epw0)).reshape(-1)
    dst_p = jnp.full((NW, EPW), N, jnp.int32)
    dst_p = dst_p.at[:, :epw0].set(dst.reshape(NW, epw0)).reshape(-1)
    ea_p = jnp.zeros((NW, EPW, DE), jnp.float32)
    ea_p = ea_p.at[:, :epw0].set(edge_attr.reshape(NW, epw0, DE))
    ea_p = ea_p.reshape(NW * EPW, DE)

    We = jnp.concatenate([W_msg1[2 * D:], W_msg2[2 * H:]], axis=1)
    be = jnp.concatenate([b_msg1, b_msg2])
    pe_both = _matmul_chunked(ea_p, We, be, bm=2048)

    Wcat1 = jnp.concatenate([W_msg1[:D], W_msg1[D:2 * D]], axis=1)
    proj1 = _matmul_chunked(hp, Wcat1, jnp.zeros((2 * H,), jnp.float32),
                            bm=512)
    pe1 = pe_both[:H // LANES]
    agg1, deg = _sc_edge_pass(src_p, dst_p, proj1, pe1,
                              zeros_h, n_pad, with_deg=True)
    h1 = _update(hp, agg1, deg, W_upd1, b_upd1, bm=512)

    Wcat2 = jnp.concatenate([W_msg2[:H], W_msg2[H:2 * H]], axis=1)
    proj2 = _matmul_chunked(h1, Wcat2, jnp.zeros((2 * H,), jnp.float32),
                            bm=512)
    pe2 = pe_both[H // LANES:]
    agg2 = _sc_edge_pass(src_p, dst_p, proj2, pe2,
                         zeros_h, n_pad, with_deg=False)
    out = _update(h1, agg2, deg, W_upd2, b_upd2, bm=512,
                  W_out=W_out, b_out=b_out)

    return out[:N]


# speedup vs baseline: 1.0512x; 1.0512x over previous
import functools

import jax
import jax.numpy as jnp
from jax import lax
from jax.experimental import pallas as pl
from jax.experimental.pallas import tpu as pltpu
from jax.experimental.pallas import tpu_sc as plsc

NC = 2
NS = 16
NW = NC * NS
LANES = 128
B = 32
EPW = 5120


def _matmul_chunked(A, W, b, bm, out_dtype=jnp.float32):
    M, K = A.shape
    Hout = W.shape[1]
    C = Hout // LANES
    assert Hout % LANES == 0 and M % bm == 0

    def body(a_ref, w_ref, b_ref, o_ref):
        o_ref[0] = (
            jnp.dot(a_ref[...], w_ref[...], preferred_element_type=jnp.float32)
            + b_ref[...]
        ).astype(out_dtype)

    return pl.pallas_call(
        body,
        grid=(M // bm, C),
        in_specs=[
            pl.BlockSpec((bm, K), lambda i, j: (i, 0)),
            pl.BlockSpec((K, LANES), lambda i, j: (0, j)),
            pl.BlockSpec((1, LANES), lambda i, j: (0, j)),
        ],
        out_specs=pl.BlockSpec((1, bm, LANES), lambda i, j: (j, i, 0)),
        out_shape=jax.ShapeDtypeStruct((C, M, LANES), out_dtype),
    )(A, W, b.reshape(1, Hout))


def _pack_i32(x):
    return jax.lax.bitcast_convert_type(
        x.reshape(*x.shape[:-1], x.shape[-1] // 2, 2), jnp.int32)


def _unpack_perm(hout):
    import numpy as np
    p = np.arange(hout)
    g, r = p // 32, p % 32
    return np.where(r % 2 == 0, 32 * g + r // 2, 32 * g + 16 + r // 2)


def _sc_edge_pass(src, dst, proj, pe, zeros_h, n_pad, with_deg):
    E = src.shape[0]
    C = pe.shape[0]
    epw = E // NW
    nb = epw // B
    nb2 = nb // 2
    rps = n_pad // NS
    assert E % NW == 0 and epw % B == 0 and nb % 2 == 0 and B % 16 == 0

    out_type = [jax.ShapeDtypeStruct((NC, C, n_pad, LANES), jnp.float32)]
    if with_deg:
        out_type.append(jax.ShapeDtypeStruct((NC, n_pad, LANES), jnp.float32))
    scratch = [
        pltpu.VMEM((epw,), jnp.int32),
        pltpu.VMEM((epw,), jnp.int32),
        pltpu.VMEM((B,), jnp.int32),
        pltpu.VMEM((B,), jnp.int32),
        pltpu.VMEM((B, LANES), jnp.float32),
        pltpu.VMEM((B, LANES), jnp.float32),
        pltpu.VMEM((B, LANES), jnp.float32),
        pltpu.VMEM((B, LANES), jnp.float32),
        pltpu.VMEM((B, LANES), jnp.float32),
        pltpu.VMEM((B, LANES), jnp.float32),
        pltpu.VMEM((B, LANES), jnp.float32),
        pltpu.VMEM((B, LANES), jnp.float32),
        pltpu.VMEM_SHARED((n_pad, LANES), jnp.float32),
    ] + [pltpu.SemaphoreType.DMA] * 8

    @functools.partial(
        pl.kernel,
        mesh=plsc.VectorSubcoreMesh(core_axis_name="c", subcore_axis_name="s"),
        out_type=out_type,
        scratch_types=scratch,
    )
    def k(src_h, dst_h, *rest):
        projs = rest[:2 * C]
        pes_h = rest[2 * C:3 * C]
        zz_h = rest[3 * C]
        rest = rest[3 * C + 1:]
        if with_deg:
            agg_h, deg_h = rest[0], rest[1]
            rest = rest[2:]
        else:
            agg_h = rest[0]
            rest = rest[1:]
        (sidx_all, didx_all, db0, db1, gs0, gs1, gd0, gd1, pe0, pe1,
         ms0, ms1, acc, gsm0, gsm1, dsm0, dsm1, psm0, psm1,
         ssm0, ssm1) = rest
        didx_buf = (db0, db1)
        gs = (gs0, gs1)
        gd = (gd0, gd1)
        pes = (pe0, pe1)
        msg = (ms0, ms1)
        gsm = (gsm0, gsm1)
        dsm = (dsm0, dsm1)
        psm = (psm0, psm1)
        ssm = (ssm0, ssm1)

        c = lax.axis_index("c")
        s = lax.axis_index("s")
        w = s * NC + c
        ebase = w * epw
        row0 = s * rps

        pltpu.sync_copy(src_h.at[pl.ds(ebase, epw)], sidx_all)
        pltpu.sync_copy(dst_h.at[pl.ds(ebase, epw)], didx_all)

        npass = C + 1 if with_deg else C
        for ch in range(npass):
            deg_pass = ch == C
            pltpu.sync_copy(zz_h.at[pl.ds(row0, rps)],
                            acc.at[pl.ds(row0, rps)])
            plsc.subcore_barrier()

            if deg_pass:
                def fill1(i, _):
                    for r in range(LANES // 16):
                        ms0[i, pl.ds(r * 16, 16)] = jnp.ones(
                            (16,), jnp.float32)
                    return 0
                lax.fori_loop(0, B, fill1, 0)

                def dbatch(kk, _):
                    for t in range(2):
                        i = 2 * kk + t
                        off = i * B
                        @pl.when(kk > 0)
                        def _():
                            pltpu.make_async_copy(
                                ms0, acc.at[pl.ds(0, B)], ssm[t]).wait()
                        for r in range(B // 16):
                            didx_buf[t][pl.ds(r * 16, 16)] = (
                                didx_all[pl.ds(off + r * 16, 16)])
                        pltpu.async_copy(
                            ms0, acc.at[didx_buf[t]], ssm[t], add=True)
                    return 0
                lax.fori_loop(0, nb2, dbatch, 0)
                for t in range(2):
                    pltpu.make_async_copy(
                        ms0, acc.at[pl.ds(0, B)], ssm[t]).wait()
            else:
                ps_h = projs[ch]
                pd_h = projs[C + ch]
                pe_ch = pes_h[ch]

                def issue(b, t):
                    off = b * B
                    pltpu.async_copy(
                        ps_h.at[sidx_all.at[pl.ds(off, B)]], gs[t], gsm[t])
                    pltpu.async_copy(
                        pd_h.at[didx_all.at[pl.ds(off, B)]], gd[t], dsm[t])
                    pltpu.async_copy(
                        pe_ch.at[pl.ds(ebase + off, B)], pes[t], psm[t])

                def wait_in(t):
                    pltpu.make_async_copy(
                        ps_h.at[pl.ds(0, B)], gs[t], gsm[t]).wait()
                    pltpu.make_async_copy(
                        pd_h.at[pl.ds(0, B)], gd[t], dsm[t]).wait()
                    pltpu.make_async_copy(
                        pe_ch.at[pl.ds(0, B)], pes[t], psm[t]).wait()

                def wait_sc(t):
                    pltpu.make_async_copy(
                        msg[t], acc.at[pl.ds(0, B)], ssm[t]).wait()

                issue(0, 0)
                issue(1, 1)

                def body2(kk, _):
                    for t in range(2):
                        b = 2 * kk + t
                        off = b * B
                        wait_in(t)
                        for r in range(B // 16):
                            didx_buf[t][pl.ds(r * 16, 16)] = (
                                didx_all[pl.ds(off + r * 16, 16)])

                        @pl.when(kk > 0)
                        def _():
                            wait_sc(t)

                        def comp(j, _):
                            for r in range(LANES // 16):
                                sl = pl.ds(r * 16, 16)
                                msg[t][j, sl] = jnp.maximum(
                                    gs[t][j, sl] + gd[t][j, sl]
                                    + pes[t][j, sl], 0.0)
                            return 0
                        lax.fori_loop(0, B, comp, 0)
                        pltpu.async_copy(
                            msg[t], acc.at[didx_buf[t]], ssm[t], add=True)

                        @pl.when(kk + 1 < nb2)
                        def _():
                            issue(b + 2, t)
                    return 0
                lax.fori_loop(0, nb2, body2, 0)
                wait_sc(0)
                wait_sc(1)

            plsc.subcore_barrier()

            for core in range(NC):
                @pl.when(c == core)
                def _():
                    if deg_pass:
                        pltpu.sync_copy(
                            acc.at[pl.ds(row0, rps)],
                            deg_h.at[core, pl.ds(row0, rps)])
                    else:
                        pltpu.sync_copy(
                            acc.at[pl.ds(row0, rps)],
                            agg_h.at[core, ch, pl.ds(row0, rps)])
            if ch + 1 < npass:
                plsc.subcore_barrier()

    proj_args = [proj[k_] for k_ in range(2 * C)]
    pe_args = [pe[k_] for k_ in range(C)]
    outs = k(src, dst, *proj_args, *pe_args, zeros_h)
    return outs if with_deg else (outs[0] if isinstance(outs, (list, tuple)) else outs)


def _update(h, agg, deg, W_u, b_u, bm, W_out=None, b_out=None):
    M, Din = h.shape
    C = agg.shape[1]
    H = C * LANES
    final = W_out is not None

    def body(*refs):
        if final:
            h_ref, agg_ref, deg_ref, wu_ref, bu_ref, wo_ref, bo_ref, o_ref = refs
        else:
            h_ref, agg_ref, deg_ref, wu_ref, bu_ref, o_ref = refs
        acc = jnp.dot(h_ref[...], wu_ref[pl.ds(0, Din), :],
                      preferred_element_type=jnp.float32)
        deg = deg_ref[0, :, 0:1] + deg_ref[1, :, 0:1]
        inv = 1.0 / jnp.maximum(deg, 1.0)
        for ch in range(C):
            a = (agg_ref[0, ch] + agg_ref[1, ch]) * inv
            acc += jnp.dot(a, wu_ref[pl.ds(Din + ch * LANES, LANES), :],
                           preferred_element_type=jnp.float32)
        hn = jnp.maximum(acc + bu_ref[...], 0.0)
        if final:
            o_ref[...] = (
                jnp.dot(hn, wo_ref[...], preferred_element_type=jnp.float32)
                + bo_ref[...]
            )
        else:
            o_ref[...] = hn

    Dtot = Din + H
    Hout = W_out.shape[1] if final else W_u.shape[1]
    in_specs = [
        pl.BlockSpec((bm, Din), lambda i: (i, 0)),
        pl.BlockSpec((2, C, bm, LANES), lambda i: (0, 0, i, 0)),
        pl.BlockSpec((2, bm, LANES), lambda i: (0, i, 0)),
        pl.BlockSpec((Dtot, W_u.shape[1]), lambda i: (0, 0)),
        pl.BlockSpec((1, W_u.shape[1]), lambda i: (0, 0)),
    ]
    args = [h, agg, deg, W_u, b_u.reshape(1, -1)]
    if final:
        in_specs += [
            pl.BlockSpec((H, Hout), lambda i: (0, 0)),
            pl.BlockSpec((1, Hout), lambda i: (0, 0)),
        ]
        args += [W_out, b_out.reshape(1, -1)]

    return pl.pallas_call(
        body,
        grid=(M // bm,),
        in_specs=in_specs,
        out_specs=pl.BlockSpec((bm, Hout), lambda i: (i, 0)),
        out_shape=jax.ShapeDtypeStruct((M, Hout), jnp.float32),
    )(*args)


@jax.jit
def kernel(x, edge_index, edge_attr,
           W_msg1, b_msg1, W_upd1, b_upd1,
           W_msg2, b_msg2, W_upd2, b_upd2,
           W_out, b_out):
    N, D = x.shape
    E = edge_index.shape[1]
    H = W_msg1.shape[1]
    C = H // LANES
    n_pad = ((N + NS * 128 - 1) // (NS * 128)) * (NS * 128)

    src = edge_index[0]
    dst = edge_index[1]
    hp = jnp.pad(x, ((0, n_pad - N), (0, 0)))
    zeros_h = jnp.zeros((n_pad, LANES), jnp.float32)

    epw0 = E // NW
    DE = edge_attr.shape[1]
    src_p = jnp.full((NW, EPW), N, jnp.int32)
    src_p = src_p.at[:, :epw0].set(src.reshape(NW, epw0)).reshape(-1)
    dst_p = jnp.full((NW, EPW), N, jnp.int32)
    dst_p = dst_p.at[:, :epw0].set(dst.reshape(NW, epw0)).reshape(-1)
    ea_p = jnp.zeros((NW, EPW, DE), jnp.float32)
    ea_p = ea_p.at[:, :epw0].set(edge_attr.reshape(NW, epw0, DE))
    ea_p = ea_p.reshape(NW * EPW, DE)

    Wcat1 = jnp.concatenate([W_msg1[:D], W_msg1[D:2 * D]], axis=1)
    proj1 = _matmul_chunked(hp, Wcat1, jnp.zeros((2 * H,), jnp.float32),
                            bm=512)
    pe1 = _matmul_chunked(ea_p, W_msg1[2 * D:], b_msg1, bm=2048)
    agg1, deg = _sc_edge_pass(src_p, dst_p, proj1, pe1,
                              zeros_h, n_pad, with_deg=True)
    h1 = _update(hp, agg1, deg, W_upd1, b_upd1, bm=512)

    Wcat2 = jnp.concatenate([W_msg2[:H], W_msg2[H:2 * H]], axis=1)
    proj2 = _matmul_chunked(h1, Wcat2, jnp.zeros((2 * H,), jnp.float32),
                            bm=512)
    pe2 = _matmul_chunked(ea_p, W_msg2[2 * H:], b_msg2, bm=2048)
    agg2 = _sc_edge_pass(src_p, dst_p, proj2, pe2,
                         zeros_h, n_pad, with_deg=False)
    out = _update(h1, agg2, deg, W_upd2, b_upd2, bm=512,
                  W_out=W_out, b_out=b_out)

    return out[:N]
